# baseline (speedup 1.0000x reference)
import functools

import jax
import jax.numpy as jnp
from jax import lax
from jax.experimental import pallas as pl
from jax.experimental.pallas import tpu as pltpu
from jax.experimental.pallas import tpu_sc as plsc

MAX_LENGTH = 40
EMBED_DIM = 64
BATCH = 16384
NUM_WORKERS = 32
BW = BATCH // NUM_WORKERS
HB = 256
STEPS = MAX_LENGTH * (BW // HB)

_mesh = plsc.VectorSubcoreMesh(core_axis_name="c", subcore_axis_name="s")


@functools.partial(
    pl.kernel,
    mesh=_mesh,
    compiler_params=pltpu.CompilerParams(
        use_tc_tiling_on_sc=False, needs_layout_passes=False),
    out_type=jax.ShapeDtypeStruct((MAX_LENGTH, EMBED_DIM, BATCH), jnp.float32),
    scratch_types=[
        pltpu.VMEM((MAX_LENGTH, BW), jnp.int32),
        pltpu.VMEM((HB, EMBED_DIM), jnp.float32),
        pltpu.VMEM((HB, EMBED_DIM), jnp.float32),
        pltpu.VMEM((EMBED_DIM, HB), jnp.float32),
        pltpu.VMEM((EMBED_DIM, HB), jnp.float32),
        pltpu.VMEM((MAX_LENGTH, EMBED_DIM), jnp.float32),
        pltpu.SemaphoreType.DMA,
        pltpu.SemaphoreType.DMA,
        pltpu.SemaphoreType.DMA,
    ],
)
def _embed(xt_hbm, tok_hbm, pos_hbm, out_hbm, idx_v, rows0, rows1,
           tbuf0, tbuf1, pos_v, sem_g0, sem_g1, sem_o):
    wid = lax.axis_index("s") * 2 + lax.axis_index("c")
    b0 = wid * BW
    pltpu.sync_copy(pos_hbm, pos_v)
    pltpu.sync_copy(xt_hbm.at[:, pl.ds(b0, BW)], idx_v)
    iota = lax.iota(jnp.int32, 16)

    rbufs = (rows0, rows1)
    tbufs = (tbuf0, tbuf1)
    gsems = (sem_g0, sem_g1)

    def fire_gathers(s_l, s_h, rbuf, sem):
        for j in range(HB // 128):
            pltpu.async_copy(
                tok_hbm.at[idx_v.at[s_l, pl.ds(s_h * HB + j * 128, 128)]],
                rbuf.at[pl.ds(j * 128, 128)], sem)

    def drain_gathers(rbuf, sem):
        pltpu.make_async_copy(tok_hbm.at[pl.ds(0, HB)], rbuf, sem).wait()

    def wait_out():
        pltpu.make_async_copy(
            tbuf0, out_hbm.at[0, :, pl.ds(0, HB)], sem_o).wait()

    def transpose_add(l, rbuf, tbuf):
        zeros = iota * 0
        lvec = zeros + l

        def d_body(d, carry):
            col = zeros + d
            pvd = plsc.load_gather(pos_v, [lvec, col])
            for g in range(HB // 16):
                row = iota + (g * 16)
                v = plsc.load_gather(rbuf, [row, col])
                tbuf[d, pl.ds(g * 16, 16)] = v + pvd
            return carry
        lax.fori_loop(0, EMBED_DIM, d_body, 0)

    fire_gathers(0, 0, rows0, sem_g0)

    def pair_body(i, carry):
        for p in range(2):
            s = 2 * i + p

            @pl.when(s >= 1)
            def _():
                wait_out()

            @pl.when(s + 1 < STEPS)
            def _():
                fire_gathers(i + p, 1 - p, rbufs[1 - p], gsems[1 - p])

            drain_gathers(rbufs[p], gsems[p])
            transpose_add(i, rbufs[p], tbufs[p])
            pltpu.async_copy(
                tbufs[p], out_hbm.at[i, :, pl.ds(b0 + p * HB, HB)], sem_o)
        return carry

    lax.fori_loop(0, STEPS // 2, pair_body, 0)
    wait_out()


def kernel(x, token_table, position_table):
    out_phys = _embed(x.T, token_table, position_table)
    return jnp.transpose(out_phys, (2, 0, 1))


# speedup vs baseline: 1.5704x; 1.5704x over previous
import functools

import jax
import jax.numpy as jnp
from jax import lax
from jax.experimental import pallas as pl
from jax.experimental.pallas import tpu as pltpu
from jax.experimental.pallas import tpu_sc as plsc

MAX_LENGTH = 40
EMBED_DIM = 64
BATCH = 16384
NUM_WORKERS = 32
BW = BATCH // NUM_WORKERS
HB = 256
STEPS = MAX_LENGTH * (BW // HB)

_mesh = plsc.VectorSubcoreMesh(core_axis_name="c", subcore_axis_name="s")


@functools.partial(
    pl.kernel,
    mesh=_mesh,
    compiler_params=pltpu.CompilerParams(
        use_tc_tiling_on_sc=False, needs_layout_passes=False),
    out_type=jax.ShapeDtypeStruct((MAX_LENGTH, EMBED_DIM, BATCH), jnp.float32),
    scratch_types=[
        pltpu.VMEM((MAX_LENGTH, BW), jnp.int32),
        pltpu.VMEM((HB, EMBED_DIM), jnp.float32),
        pltpu.VMEM((HB, EMBED_DIM), jnp.float32),
        pltpu.VMEM((EMBED_DIM, HB + 1), jnp.float32),
        pltpu.VMEM((EMBED_DIM, HB + 1), jnp.float32),
        pltpu.VMEM((MAX_LENGTH, EMBED_DIM), jnp.float32),
        pltpu.SemaphoreType.DMA,
        pltpu.SemaphoreType.DMA,
        pltpu.SemaphoreType.DMA,
    ],
)
def _embed(xt_hbm, tok_hbm, pos_hbm, out_hbm, idx_v, rows0, rows1,
           tbuf0, tbuf1, pos_v, sem_g0, sem_g1, sem_o):
    wid = lax.axis_index("s") * 2 + lax.axis_index("c")
    b0 = wid * BW
    pltpu.sync_copy(pos_hbm, pos_v)
    pltpu.sync_copy(xt_hbm.at[:, pl.ds(b0, BW)], idx_v)
    iota = lax.iota(jnp.int32, 16)

    rbufs = (rows0, rows1)
    tbufs = (tbuf0, tbuf1)
    gsems = (sem_g0, sem_g1)

    def fire_gathers(s_l, s_h, rbuf, sem):
        for j in range(HB // 128):
            pltpu.async_copy(
                tok_hbm.at[idx_v.at[s_l, pl.ds(s_h * HB + j * 128, 128)]],
                rbuf.at[pl.ds(j * 128, 128)], sem)

    def drain_gathers(rbuf, sem):
        pltpu.make_async_copy(tok_hbm.at[pl.ds(0, HB)], rbuf, sem).wait()

    def wait_out():
        pltpu.make_async_copy(
            tbuf0.at[:, pl.ds(0, HB)], out_hbm.at[0, :, pl.ds(0, HB)],
            sem_o).wait()

    rowks = [iota + 16 * k for k in range(EMBED_DIM // 16)]
    zeros = iota * 0

    def transpose_add(l, rbuf, tbuf):
        pvs = [pos_v[l, pl.ds(16 * k, 16)] for k in range(EMBED_DIM // 16)]

        def r_body(r4, carry):
            for u in range(4):
                r = r4 * 4 + u
                col = zeros + r
                for k in range(EMBED_DIM // 16):
                    v = rbuf[r, pl.ds(16 * k, 16)] + pvs[k]
                    plsc.store_scatter(tbuf, [rowks[k], col], v)
            return carry
        lax.fori_loop(0, HB // 4, r_body, 0)

    fire_gathers(0, 0, rows0, sem_g0)

    def pair_body(i, carry):
        for p in range(2):
            s = 2 * i + p

            @pl.when(s >= 1)
            def _():
                wait_out()

            @pl.when(s + 1 < STEPS)
            def _():
                fire_gathers(i + p, 1 - p, rbufs[1 - p], gsems[1 - p])

            drain_gathers(rbufs[p], gsems[p])
            transpose_add(i, rbufs[p], tbufs[p])
            pltpu.async_copy(
                tbufs[p].at[:, pl.ds(0, HB)],
                out_hbm.at[i, :, pl.ds(b0 + p * HB, HB)], sem_o)
        return carry

    lax.fori_loop(0, STEPS // 2, pair_body, 0)
    wait_out()


def kernel(x, token_table, position_table):
    out_phys = _embed(x.T, token_table, position_table)
    return jnp.transpose(out_phys, (2, 0, 1))


# speedup vs baseline: 1.8284x; 1.1643x over previous
import functools

import jax
import jax.numpy as jnp
from jax import lax
from jax.experimental import pallas as pl
from jax.experimental.pallas import tpu as pltpu
from jax.experimental.pallas import tpu_sc as plsc

MAX_LENGTH = 40
EMBED_DIM = 64
BATCH = 16384
NUM_WORKERS = 32
BT = BATCH // 128
TPW = BT // NUM_WORKERS
HB = 256
STEPS = MAX_LENGTH * 2
TP = HB + 1

_mesh = plsc.VectorSubcoreMesh(core_axis_name="c", subcore_axis_name="s")


@functools.partial(
    pl.kernel,
    mesh=_mesh,
    compiler_params=pltpu.CompilerParams(
        use_tc_tiling_on_sc=False, needs_layout_passes=False),
    out_type=jax.ShapeDtypeStruct(
        (MAX_LENGTH, EMBED_DIM // 8, BT, 8, 128), jnp.float32),
    scratch_types=[
        pltpu.VMEM((MAX_LENGTH // 8, TPW, 8, 128), jnp.int32),
        pltpu.VMEM((HB, EMBED_DIM), jnp.float32),
        pltpu.VMEM((HB, EMBED_DIM), jnp.float32),
        pltpu.VMEM((EMBED_DIM, TP), jnp.float32),
        pltpu.VMEM((EMBED_DIM, TP), jnp.float32),
        pltpu.VMEM((MAX_LENGTH, EMBED_DIM), jnp.float32),
        pltpu.SemaphoreType.DMA,
        pltpu.SemaphoreType.DMA,
        pltpu.SemaphoreType.DMA,
    ],
)
def _embed(x4_hbm, tok_hbm, pos_hbm, out_hbm, idx_v, rows0, rows1,
           tbuf0, tbuf1, pos_v, sem_g0, sem_g1, sem_o):
    wid = lax.axis_index("s") * 2 + lax.axis_index("c")
    tb0 = wid * TPW
    pltpu.sync_copy(pos_hbm, pos_v)
    pltpu.sync_copy(x4_hbm.at[:, pl.ds(tb0, TPW)], idx_v)
    iota = lax.iota(jnp.int32, 16)

    rbufs = (rows0, rows1)
    tbufs = (tbuf0, tbuf1)
    gsems = (sem_g0, sem_g1)

    def fire_gathers(s_l, s_h, rbuf, sem):
        for j in range(2):
            pltpu.async_copy(
                tok_hbm.at[idx_v.at[s_l >> 3, 2 * s_h + j, s_l & 7]],
                rbuf.at[pl.ds(j * 128, 128)], sem)

    def drain_gathers(rbuf, sem):
        pltpu.make_async_copy(tok_hbm.at[pl.ds(0, HB)], rbuf, sem).wait()

    def wait_out():
        for _ in range(16):
            pltpu.make_async_copy(
                tbuf0.at[pl.ds(0, 8), pl.ds(0, 128)],
                out_hbm.at[0, 0, 0], sem_o).wait()

    rowks = [iota + 16 * k for k in range(EMBED_DIM // 16)]
    zeros = iota * 0

    def transpose_add(l, rbuf, tbuf):
        pvs = [pos_v[l, pl.ds(16 * k, 16)] for k in range(EMBED_DIM // 16)]

        def r_body(r4, carry):
            for u in range(4):
                r = r4 * 4 + u
                col = zeros + r
                for k in range(EMBED_DIM // 16):
                    v = rbuf[r, pl.ds(16 * k, 16)] + pvs[k]
                    plsc.store_scatter(tbuf, [rowks[k], col], v)
            return carry
        lax.fori_loop(0, HB // 4, r_body, 0)

    fire_gathers(0, 0, rows0, sem_g0)

    def pair_body(i, carry):
        for p in range(2):
            s = 2 * i + p

            @pl.when(s >= 1)
            def _():
                wait_out()

            @pl.when(s + 1 < STEPS)
            def _():
                fire_gathers(i + p, 1 - p, rbufs[1 - p], gsems[1 - p])

            drain_gathers(rbufs[p], gsems[p])
            transpose_add(i, rbufs[p], tbufs[p])
            for e in range(EMBED_DIM // 8):
                for j in range(2):
                    pltpu.async_copy(
                        tbufs[p].at[pl.ds(8 * e, 8), pl.ds(128 * j, 128)],
                        out_hbm.at[i, e, tb0 + 2 * p + j], sem_o)
        return carry

    lax.fori_loop(0, STEPS // 2, pair_body, 0)
    wait_out()


def kernel(x, token_table, position_table):
    x4 = x.T.reshape(MAX_LENGTH // 8, 8, BT, 128).transpose(0, 2, 1, 3)
    out5 = _embed(x4, token_table, position_table)
    return out5.transpose(2, 4, 0, 1, 3).reshape(BATCH, MAX_LENGTH, EMBED_DIM)


# speedup vs baseline: 1.8395x; 1.0061x over previous
import functools

import jax
import jax.numpy as jnp
from jax import lax
from jax.experimental import pallas as pl
from jax.experimental.pallas import tpu as pltpu
from jax.experimental.pallas import tpu_sc as plsc

MAX_LENGTH = 40
EMBED_DIM = 64
BATCH = 16384
NUM_WORKERS = 32
BT = BATCH // 128
TPW = BT // NUM_WORKERS
HB = 256
STEPS = MAX_LENGTH * 2
TP = HB + 1

_mesh = plsc.VectorSubcoreMesh(core_axis_name="c", subcore_axis_name="s")


@functools.partial(
    pl.kernel,
    mesh=_mesh,
    compiler_params=pltpu.CompilerParams(
        use_tc_tiling_on_sc=False, needs_layout_passes=False),
    out_type=jax.ShapeDtypeStruct(
        (MAX_LENGTH, EMBED_DIM // 8, BT, 8, 128), jnp.float32),
    scratch_types=[
        pltpu.VMEM((MAX_LENGTH // 8, TPW, 8, 128), jnp.int32),
        pltpu.VMEM((HB, EMBED_DIM), jnp.float32),
        pltpu.VMEM((HB, EMBED_DIM), jnp.float32),
        pltpu.VMEM((EMBED_DIM, TP), jnp.float32),
        pltpu.VMEM((EMBED_DIM, TP), jnp.float32),
        pltpu.VMEM((MAX_LENGTH, EMBED_DIM), jnp.float32),
        pltpu.SemaphoreType.DMA,
        pltpu.SemaphoreType.DMA,
        pltpu.SemaphoreType.DMA,
    ],
)
def _embed(tok_hbm, x4_hbm, pos_hbm, out_hbm, idx_v, rows0, rows1,
           tbuf0, tbuf1, pos_v, sem_g0, sem_g1, sem_o):
    wid = lax.axis_index("s") * 2 + lax.axis_index("c")
    tb0 = wid * TPW
    pltpu.sync_copy(pos_hbm, pos_v)
    pltpu.sync_copy(x4_hbm.at[:, pl.ds(tb0, TPW)], idx_v)
    iota = lax.iota(jnp.int32, 16)

    rbufs = (rows0, rows1)
    tbufs = (tbuf0, tbuf1)
    gsems = (sem_g0, sem_g1)

    def fire_gathers(s_l, s_h, rbuf, sem):
        for j in range(2):
            pltpu.async_copy(
                tok_hbm.at[idx_v.at[s_l >> 3, 2 * s_h + j, s_l & 7]],
                rbuf.at[pl.ds(j * 128, 128)], sem)

    def drain_gathers(rbuf, sem):
        pltpu.make_async_copy(tok_hbm.at[pl.ds(0, HB)], rbuf, sem).wait()

    def wait_out():
        pltpu.make_async_copy(
            rows0, tok_hbm.at[pl.ds(0, HB)], sem_o).wait()

    rowks = [iota + 16 * k for k in range(EMBED_DIM // 16)]
    zeros = iota * 0

    def transpose_add(l, rbuf, tbuf):
        pvs = [pos_v[l, pl.ds(16 * k, 16)] for k in range(EMBED_DIM // 16)]

        def r_body(r8, carry):
            for u in range(8):
                r = r8 * 8 + u
                col = zeros + r
                for k in range(EMBED_DIM // 16):
                    v = rbuf[r, pl.ds(16 * k, 16)] + pvs[k]
                    plsc.store_scatter(tbuf, [rowks[k], col], v)
            return carry
        lax.fori_loop(0, HB // 8, r_body, 0)

    fire_gathers(0, 0, rows0, sem_g0)

    def pair_body(i, carry):
        for p in range(2):
            s = 2 * i + p

            @pl.when(s + 1 < STEPS)
            def _():
                fire_gathers(i + p, 1 - p, rbufs[1 - p], gsems[1 - p])

            @pl.when(s >= 1)
            def _():
                wait_out()

            drain_gathers(rbufs[p], gsems[p])
            transpose_add(i, rbufs[p], tbufs[p])
            for e in range(EMBED_DIM // 8):
                for j in range(2):
                    pltpu.async_copy(
                        tbufs[p].at[pl.ds(8 * e, 8), pl.ds(128 * j, 128)],
                        out_hbm.at[i, e, tb0 + 2 * p + j], sem_o)
        return carry

    lax.fori_loop(0, STEPS // 2, pair_body, 0)
    wait_out()


def kernel(x, token_table, position_table):
    x4 = x.T.reshape(MAX_LENGTH // 8, 8, BT, 128).transpose(0, 2, 1, 3)
    out5 = _embed(token_table, x4, position_table)
    return out5.transpose(2, 4, 0, 1, 3).reshape(BATCH, MAX_LENGTH, EMBED_DIM)


# speedup vs baseline: 1.8428x; 1.0018x over previous
import functools

import jax
import jax.numpy as jnp
from jax import lax
from jax.experimental import pallas as pl
from jax.experimental.pallas import tpu as pltpu
from jax.experimental.pallas import tpu_sc as plsc

MAX_LENGTH = 40
EMBED_DIM = 64
BATCH = 16384
NUM_WORKERS = 32
BT = BATCH // 128
TPW = BT // NUM_WORKERS
HB = 256
STEPS = MAX_LENGTH * 2
TP = HB + 1

_mesh = plsc.VectorSubcoreMesh(core_axis_name="c", subcore_axis_name="s")


@functools.partial(
    pl.kernel,
    mesh=_mesh,
    compiler_params=pltpu.CompilerParams(
        use_tc_tiling_on_sc=False, needs_layout_passes=False),
    out_type=jax.ShapeDtypeStruct(
        (MAX_LENGTH, EMBED_DIM // 8, BT, 8, 128), jnp.float32),
    scratch_types=[
        pltpu.VMEM((MAX_LENGTH // 8, TPW, 8, 128), jnp.int32),
        pltpu.VMEM((HB, EMBED_DIM), jnp.float32),
        pltpu.VMEM((HB, EMBED_DIM), jnp.float32),
        pltpu.VMEM((EMBED_DIM // 8, 2, 8, 129), jnp.float32),
        pltpu.VMEM((EMBED_DIM // 8, 2, 8, 129), jnp.float32),
        pltpu.VMEM((MAX_LENGTH, EMBED_DIM), jnp.float32),
        pltpu.SemaphoreType.DMA,
        pltpu.SemaphoreType.DMA,
        pltpu.SemaphoreType.DMA,
    ],
)
def _embed(tok_hbm, x4_hbm, pos_hbm, out_hbm, idx_v, rows0, rows1,
           tbuf0, tbuf1, pos_v, sem_g0, sem_g1, sem_o):
    wid = lax.axis_index("s") * 2 + lax.axis_index("c")
    tb0 = wid * TPW
    pltpu.sync_copy(pos_hbm, pos_v)
    pltpu.sync_copy(x4_hbm.at[:, pl.ds(tb0, TPW)], idx_v)
    iota = lax.iota(jnp.int32, 16)

    rbufs = (rows0, rows1)
    tbufs = (tbuf0, tbuf1)
    gsems = (sem_g0, sem_g1)

    def fire_gathers(s_l, s_h, rbuf, sem):
        for j in range(2):
            pltpu.async_copy(
                tok_hbm.at[idx_v.at[s_l >> 3, 2 * s_h + j, s_l & 7]],
                rbuf.at[pl.ds(j * 128, 128)], sem)

    def drain_gathers(rbuf, sem):
        pltpu.make_async_copy(tok_hbm.at[pl.ds(0, HB)], rbuf, sem).wait()

    def wait_out():
        pltpu.make_async_copy(
            rows0, tok_hbm.at[pl.ds(0, HB)], sem_o).wait()

    evecs = [(iota >> 3) + 2 * k for k in range(EMBED_DIM // 16)]
    svec = iota & 7
    zeros = iota * 0

    def transpose_add(l, rbuf, tbuf):
        pvs = [pos_v[l, pl.ds(16 * k, 16)] for k in range(EMBED_DIM // 16)]

        def r_body(r8, carry):
            for u in range(8):
                r = r8 * 8 + u
                jvec = zeros + (r >> 7)
                cvec = zeros + (r & 127)
                for k in range(EMBED_DIM // 16):
                    v = rbuf[r, pl.ds(16 * k, 16)] + pvs[k]
                    plsc.store_scatter(tbuf, [evecs[k], jvec, svec, cvec], v)
            return carry
        lax.fori_loop(0, HB // 8, r_body, 0)

    fire_gathers(0, 0, rows0, sem_g0)

    def pair_body(i, carry):
        for p in range(2):
            s = 2 * i + p

            @pl.when(s + 1 < STEPS)
            def _():
                fire_gathers(i + p, 1 - p, rbufs[1 - p], gsems[1 - p])

            @pl.when(s >= 1)
            def _():
                wait_out()

            drain_gathers(rbufs[p], gsems[p])
            transpose_add(i, rbufs[p], tbufs[p])
            pltpu.async_copy(
                tbufs[p].at[:, :, :, pl.ds(0, 128)],
                out_hbm.at[i, :, pl.ds(tb0 + 2 * p, 2)], sem_o)
        return carry

    lax.fori_loop(0, STEPS // 2, pair_body, 0)
    wait_out()


def kernel(x, token_table, position_table):
    x4 = x.T.reshape(MAX_LENGTH // 8, 8, BT, 128).transpose(0, 2, 1, 3)
    out5 = _embed(token_table, x4, position_table)
    return out5.transpose(2, 4, 0, 1, 3).reshape(BATCH, MAX_LENGTH, EMBED_DIM)
